# baseline (speedup 1.0000x reference)
import functools

import jax
import jax.numpy as jnp
from jax import lax
from jax.experimental import pallas as pl
from jax.experimental.pallas import tpu as pltpu
from jax.experimental.pallas import tpu_sc as plsc

_info = plsc.get_sparse_core_info()
_NC, _NS = _info.num_cores, _info.num_subcores
_NW = _NC * _NS

_CHUNK_ROWS = 16
_NBUF = 3


def _make_copy_kernel(seq_len: int, d_model: int):
    rows_per_w = seq_len // _NW
    n_chunks = rows_per_w // _CHUNK_ROWS
    mesh = plsc.VectorSubcoreMesh(core_axis_name="c", subcore_axis_name="s")

    @functools.partial(
        pl.kernel,
        mesh=mesh,
        out_type=jax.ShapeDtypeStruct((seq_len, d_model), jnp.float32),
        scratch_types=(
            [pltpu.VMEM((_CHUNK_ROWS, d_model), jnp.float32) for _ in range(_NBUF)]
            + [pltpu.SemaphoreType.DMA for _ in range(2 * _NBUF)]
        ),
    )
    def copy_rows(table_hbm, out_hbm, *scratch):
        bufs = list(scratch[:_NBUF])
        lsem = list(scratch[_NBUF : 2 * _NBUF])
        ssem = list(scratch[2 * _NBUF :])
        wid = lax.axis_index("s") * _NC + lax.axis_index("c")
        base = wid * rows_per_w

        def src(i):
            return table_hbm.at[pl.ds(base + i * _CHUNK_ROWS, _CHUNK_ROWS)]

        def dst(i):
            return out_hbm.at[pl.ds(base + i * _CHUNK_ROWS, _CHUNK_ROWS)]

        loads = [None] * n_chunks
        stores = [None] * n_chunks
        for j in range(min(_NBUF - 1, n_chunks)):
            loads[j] = pltpu.async_copy(src(j), bufs[j % _NBUF], lsem[j % _NBUF])
        for i in range(n_chunks):
            b = i % _NBUF
            loads[i].wait()
            stores[i] = pltpu.async_copy(bufs[b], dst(i), ssem[b])
            j = i + _NBUF - 1
            if j < n_chunks:
                if i >= 1:
                    stores[i - 1].wait()
                loads[j] = pltpu.async_copy(src(j), bufs[j % _NBUF], lsem[j % _NBUF])
        for i in range(max(0, n_chunks - _NBUF), n_chunks):
            stores[i].wait()

    return copy_rows


@jax.jit
def kernel(inputs, pos_table):
    seq_len = inputs.shape[1]
    return _make_copy_kernel(seq_len, pos_table.shape[1])(pos_table)


# speedup vs baseline: 1.0387x; 1.0387x over previous
import functools

import jax
import jax.numpy as jnp
from jax import lax
from jax.experimental import pallas as pl
from jax.experimental.pallas import tpu as pltpu
from jax.experimental.pallas import tpu_sc as plsc

_info = plsc.get_sparse_core_info()
_NC, _NS = _info.num_cores, _info.num_subcores
_NW = _NC * _NS

_CHUNK_ROWS = 8
_NBUF = 6


def _make_copy_kernel(seq_len: int, d_model: int):
    rows_per_w = seq_len // _NW
    n_chunks = rows_per_w // _CHUNK_ROWS
    mesh = plsc.VectorSubcoreMesh(core_axis_name="c", subcore_axis_name="s")

    @functools.partial(
        pl.kernel,
        mesh=mesh,
        out_type=jax.ShapeDtypeStruct((seq_len, d_model), jnp.float32),
        scratch_types=(
            [pltpu.VMEM((_CHUNK_ROWS, d_model), jnp.float32) for _ in range(_NBUF)]
            + [pltpu.SemaphoreType.DMA for _ in range(2 * _NBUF)]
        ),
    )
    def copy_rows(table_hbm, out_hbm, *scratch):
        bufs = list(scratch[:_NBUF])
        lsem = list(scratch[_NBUF : 2 * _NBUF])
        ssem = list(scratch[2 * _NBUF :])
        wid = lax.axis_index("s") * _NC + lax.axis_index("c")
        base = wid * rows_per_w

        def src(i):
            return table_hbm.at[pl.ds(base + i * _CHUNK_ROWS, _CHUNK_ROWS)]

        def dst(i):
            return out_hbm.at[pl.ds(base + i * _CHUNK_ROWS, _CHUNK_ROWS)]

        loads = [None] * n_chunks
        stores = [None] * n_chunks
        for j in range(min(_NBUF - 1, n_chunks)):
            loads[j] = pltpu.async_copy(src(j), bufs[j % _NBUF], lsem[j % _NBUF])
        for i in range(n_chunks):
            b = i % _NBUF
            loads[i].wait()
            stores[i] = pltpu.async_copy(bufs[b], dst(i), ssem[b])
            j = i + _NBUF - 1
            if j < n_chunks:
                if i >= 1:
                    stores[i - 1].wait()
                loads[j] = pltpu.async_copy(src(j), bufs[j % _NBUF], lsem[j % _NBUF])
        for i in range(max(0, n_chunks - _NBUF), n_chunks):
            stores[i].wait()

    return copy_rows


@jax.jit
def kernel(inputs, pos_table):
    seq_len = inputs.shape[1]
    return _make_copy_kernel(seq_len, pos_table.shape[1])(pos_table)
